# Initial kernel scaffold; baseline (speedup 1.0000x reference)
import jax
import jax.numpy as jnp
from jax.experimental import pallas as pl


def kernel(h, pos, edge_index, emb, edge_w1, edge_b1, edge_w2, edge_b2, coord_w1, coord_b1, coord_w2, coord_b2, node_w1, node_b1, node_w2, node_b2):
    raise NotImplementedError("write your pallas kernel here")


# speedup vs baseline: 1.0080x; 1.0080x over previous
import functools

import jax
import jax.numpy as jnp
from jax.experimental import pallas as pl
from jax.experimental.pallas import tpu as pltpu

H = 128
EBLK = 2000
NBLK = 1000


def _silu(v):
    return v * jax.nn.sigmoid(v)


def _edge_body(t_ref, w2_ref, b2_ref, cw1_ref, cb1_ref, cw2_ref, cb2_ref,
               m_ref, c_ref):
    m1 = _silu(t_ref[...])
    m = _silu(jnp.dot(m1, w2_ref[...], preferred_element_type=jnp.float32)
              + b2_ref[...])
    cc = _silu(jnp.dot(m, cw1_ref[...], preferred_element_type=jnp.float32)
               + cb1_ref[...])
    c = jnp.dot(cc, cw2_ref[...], preferred_element_type=jnp.float32) + cb2_ref[...]
    m_ref[...] = m
    c_ref[...] = c


@jax.jit
def _edge_mlp(t, w2, b2, cw1, cb1, cw2, cb2):
    e = t.shape[0]
    grid = e // EBLK
    full = pl.BlockSpec((H, H), lambda i: (0, 0))
    row = pl.BlockSpec((1, H), lambda i: (0, 0))
    return pl.pallas_call(
        _edge_body,
        grid=(grid,),
        in_specs=[
            pl.BlockSpec((EBLK, H), lambda i: (i, 0)),
            full, row, full, row,
            pl.BlockSpec((H, 1), lambda i: (0, 0)),
            pl.BlockSpec((1, 1), lambda i: (0, 0)),
        ],
        out_specs=[
            pl.BlockSpec((EBLK, H), lambda i: (i, 0)),
            pl.BlockSpec((EBLK, 1), lambda i: (i, 0)),
        ],
        out_shape=[
            jax.ShapeDtypeStruct((e, H), jnp.float32),
            jax.ShapeDtypeStruct((e, 1), jnp.float32),
        ],
    )(t, w2, b2.reshape(1, H), cw1, cb1.reshape(1, H), cw2, cb2.reshape(1, 1))


def _b16(v):
    return v.astype(jnp.bfloat16)


def _dot16(a, b):
    return jnp.dot(_b16(a), _b16(b), preferred_element_type=jnp.float32)


def _node_body(x_ref, agg_ref, nw1a_ref, nw1b_ref, nb1_ref, nw2_ref, nb2_ref,
               w1a_ref, b1_ref, w1b_ref, xn_ref, a_ref, b_ref):
    x = x_ref[...]
    u = _silu(_dot16(x, nw1a_ref[...]) + _dot16(agg_ref[...], nw1b_ref[...])
              + nb1_ref[...])
    xn = x + _dot16(u, nw2_ref[...]) + nb2_ref[...]
    xn_ref[...] = xn
    a_ref[...] = _dot16(xn, w1a_ref[...]) + b1_ref[...]
    b_ref[...] = _dot16(xn, w1b_ref[...])


@jax.jit
def _node_update(x, agg, nw1, nb1, nw2, nb2, w1a, b1, w1b):
    n = x.shape[0]
    grid = n // NBLK
    full = pl.BlockSpec((H, H), lambda i: (0, 0))
    row = pl.BlockSpec((1, H), lambda i: (0, 0))
    blk = pl.BlockSpec((NBLK, H), lambda i: (i, 0))
    return pl.pallas_call(
        _node_body,
        grid=(grid,),
        in_specs=[blk, blk, full, full, row, full, row, full, row, full],
        out_specs=[blk, blk, blk],
        out_shape=[
            jax.ShapeDtypeStruct((n, H), jnp.float32),
            jax.ShapeDtypeStruct((n, H), jnp.float32),
            jax.ShapeDtypeStruct((n, H), jnp.float32),
        ],
    )(x, agg, nw1[:H], nw1[H:], nb1.reshape(1, H), nw2, nb2.reshape(1, H),
      w1a, b1.reshape(1, H), w1b)


def _ab_body(x_ref, w1a_ref, b1_ref, w1b_ref, a_ref, b_ref):
    x = x_ref[...]
    a_ref[...] = _dot16(x, w1a_ref[...]) + b1_ref[...]
    b_ref[...] = _dot16(x, w1b_ref[...])


@jax.jit
def _ab_precompute(x, w1a, b1, w1b):
    n = x.shape[0]
    grid = n // NBLK
    full = pl.BlockSpec((H, H), lambda i: (0, 0))
    row = pl.BlockSpec((1, H), lambda i: (0, 0))
    blk = pl.BlockSpec((NBLK, H), lambda i: (i, 0))
    return pl.pallas_call(
        _ab_body,
        grid=(grid,),
        in_specs=[blk, full, row, full],
        out_specs=[blk, blk],
        out_shape=[
            jax.ShapeDtypeStruct((n, H), jnp.float32),
            jax.ShapeDtypeStruct((n, H), jnp.float32),
        ],
    )(x, w1a, b1.reshape(1, H), w1b)


def _round_bf16(v):
    u = jax.lax.bitcast_convert_type(v, jnp.uint32)
    r = (u + jnp.uint32(0x7FFF) + ((u >> 16) & jnp.uint32(1))) & jnp.uint32(0xFFFF0000)
    return jax.lax.bitcast_convert_type(r, jnp.float32)


def kernel(h, pos, edge_index, emb, edge_w1, edge_b1, edge_w2, edge_b2,
           coord_w1, coord_b1, coord_w2, coord_b2, node_w1, node_b1,
           node_w2, node_b2):
    L = edge_w1.shape[0]
    src = edge_index[0]
    dst = edge_index[1]
    n = pos.shape[0]
    deg = jax.ops.segment_sum(jnp.ones((src.shape[0], 1), pos.dtype), src,
                              num_segments=n)
    deg = jnp.maximum(deg, 1.0)
    x = emb[h]
    pos0 = pos
    a, b = _ab_precompute(x, edge_w1[0][:H], edge_b1[0], edge_w1[0][H:2 * H])
    for l in range(L):
        wd = _round_bf16(edge_w1[l][2 * H])
        rel = pos[src] - pos[dst]
        d2 = jnp.sum(rel * rel, axis=-1, keepdims=True)
        t = a[src] + b[dst] + _round_bf16(d2) * wd[None, :]
        m, c = _edge_mlp(t, edge_w2[l], edge_b2[l], coord_w1[l], coord_b1[l],
                         coord_w2[l], coord_b2[l])
        pos = pos + jax.ops.segment_sum(rel * c, src, num_segments=n) / deg
        agg = jax.ops.segment_sum(m, src, num_segments=n)
        if l + 1 < L:
            u = _silu(jnp.concatenate([x, agg], axis=-1) @ node_w1[l] + node_b1[l])
            x = x + (u @ node_w2[l] + node_b2[l])
            a, b = _ab_precompute(x, edge_w1[l + 1][:H], edge_b1[l + 1],
                                  edge_w1[l + 1][H:2 * H])
    return pos - pos0


# speedup vs baseline: 1.2471x; 1.2372x over previous
import functools

import jax
import jax.numpy as jnp
from jax import lax
from jax.experimental import pallas as pl
from jax.experimental.pallas import tpu as pltpu
from jax.experimental.pallas import tpu_sc as plsc

H = 128
EBLK = 2000
NBLK = 1000
SC_EBLK = 80


def _silu(v):
    return v * jax.nn.sigmoid(v)


def _edge_body(t_ref, w2_ref, b2_ref, cw1_ref, cb1_ref, cw2_ref, cb2_ref,
               m_ref, c_ref):
    m1 = _silu(t_ref[...])
    m = _silu(jnp.dot(m1, w2_ref[...], preferred_element_type=jnp.float32)
              + b2_ref[...])
    cc = _silu(jnp.dot(m, cw1_ref[...], preferred_element_type=jnp.float32)
               + cb1_ref[...])
    c = jnp.dot(cc, cw2_ref[...], preferred_element_type=jnp.float32) + cb2_ref[...]
    m_ref[...] = m
    c_ref[...] = c


@jax.jit
def _edge_mlp(t, w2, b2, cw1, cb1, cw2, cb2):
    e = t.shape[0]
    grid = e // EBLK
    full = pl.BlockSpec((H, H), lambda i: (0, 0))
    row = pl.BlockSpec((1, H), lambda i: (0, 0))
    return pl.pallas_call(
        _edge_body,
        grid=(grid,),
        in_specs=[
            pl.BlockSpec((EBLK, H), lambda i: (i, 0)),
            full, row, full, row,
            pl.BlockSpec((H, 1), lambda i: (0, 0)),
            pl.BlockSpec((1, 1), lambda i: (0, 0)),
        ],
        out_specs=[
            pl.BlockSpec((EBLK, H), lambda i: (i, 0)),
            pl.BlockSpec((EBLK, 1), lambda i: (i, 0)),
        ],
        out_shape=[
            jax.ShapeDtypeStruct((e, H), jnp.float32),
            jax.ShapeDtypeStruct((e, 1), jnp.float32),
        ],
    )(t, w2, b2.reshape(1, H), cw1, cb1.reshape(1, H), cw2, cb2.reshape(1, 1))


def _b16(v):
    return v.astype(jnp.bfloat16)


def _dot16(a, b):
    return jnp.dot(_b16(a), _b16(b), preferred_element_type=jnp.float32)


def _node_body(x_ref, agg_ref, nw1a_ref, nw1b_ref, nb1_ref, nw2_ref, nb2_ref,
               w1a_ref, b1_ref, w1b_ref, xn_ref, a_ref, b_ref):
    x = x_ref[...]
    u = _silu(_dot16(x, nw1a_ref[...]) + _dot16(agg_ref[...], nw1b_ref[...])
              + nb1_ref[...])
    xn = x + _dot16(u, nw2_ref[...]) + nb2_ref[...]
    xn_ref[...] = xn
    a_ref[...] = _dot16(xn, w1a_ref[...]) + b1_ref[...]
    b_ref[...] = _dot16(xn, w1b_ref[...])


@jax.jit
def _node_update(x, agg, nw1, nb1, nw2, nb2, w1a, b1, w1b):
    n = x.shape[0]
    grid = n // NBLK
    full = pl.BlockSpec((H, H), lambda i: (0, 0))
    row = pl.BlockSpec((1, H), lambda i: (0, 0))
    blk = pl.BlockSpec((NBLK, H), lambda i: (i, 0))
    return pl.pallas_call(
        _node_body,
        grid=(grid,),
        in_specs=[blk, blk, full, full, row, full, row, full, row, full],
        out_specs=[blk, blk, blk],
        out_shape=[
            jax.ShapeDtypeStruct((n, H), jnp.float32),
            jax.ShapeDtypeStruct((n, H), jnp.float32),
            jax.ShapeDtypeStruct((n, H), jnp.float32),
        ],
    )(x, agg, nw1[:H], nw1[H:], nb1.reshape(1, H), nw2, nb2.reshape(1, H),
      w1a, b1.reshape(1, H), w1b)


def _ab_body(x_ref, w1a_ref, b1_ref, w1b_ref, a_ref, b_ref):
    x = x_ref[...]
    a_ref[...] = _dot16(x, w1a_ref[...]) + b1_ref[...]
    b_ref[...] = _dot16(x, w1b_ref[...])


@jax.jit
def _ab_precompute(x, w1a, b1, w1b):
    n = x.shape[0]
    grid = n // NBLK
    full = pl.BlockSpec((H, H), lambda i: (0, 0))
    row = pl.BlockSpec((1, H), lambda i: (0, 0))
    blk = pl.BlockSpec((NBLK, H), lambda i: (i, 0))
    return pl.pallas_call(
        _ab_body,
        grid=(grid,),
        in_specs=[blk, full, row, full],
        out_specs=[blk, blk],
        out_shape=[
            jax.ShapeDtypeStruct((n, H), jnp.float32),
            jax.ShapeDtypeStruct((n, H), jnp.float32),
        ],
    )(x, w1a, b1.reshape(1, H), w1b)


@functools.cache
def _make_sc_gather(e_total):
    info = plsc.get_sparse_core_info()
    nw = info.num_cores * info.num_subcores
    per_w = e_total // nw
    chunks = per_w // SC_EBLK
    mesh = plsc.VectorSubcoreMesh(core_axis_name="c", subcore_axis_name="s")

    @functools.partial(
        pl.kernel, mesh=mesh,
        out_type=jax.ShapeDtypeStruct((e_total, H), jnp.float32),
        scratch_types=[
            pltpu.VMEM((SC_EBLK,), jnp.int32),
            pltpu.VMEM((SC_EBLK,), jnp.int32),
            pltpu.VMEM((SC_EBLK,), jnp.float32),
            pltpu.VMEM((SC_EBLK, H), jnp.float32),
            pltpu.VMEM((SC_EBLK, H), jnp.float32),
            pltpu.VMEM((SC_EBLK, H), jnp.float32),
            pltpu.VMEM((H,), jnp.float32),
            pltpu.SemaphoreType.DMA,
        ],
    )
    def k(a_hbm, b_hbm, src_hbm, dst_hbm, dd_hbm, wd_hbm, t_hbm,
          idxs, idxd, ddb, arows, brows, trows, wdv, sem):
        wid = lax.axis_index("s") * info.num_cores + lax.axis_index("c")
        base0 = wid * per_w
        pltpu.sync_copy(wd_hbm, wdv)
        wvs = [wdv[pl.ds(16 * j, 16)] for j in range(H // 16)]

        def chunk_body(c, carry):
            base = base0 + c * SC_EBLK
            pltpu.sync_copy(src_hbm.at[pl.ds(base, SC_EBLK)], idxs)
            pltpu.sync_copy(dst_hbm.at[pl.ds(base, SC_EBLK)], idxd)
            pltpu.sync_copy(dd_hbm.at[pl.ds(base, SC_EBLK)], ddb)
            ca = pltpu.async_copy(a_hbm.at[idxs], arows, sem)
            cb = pltpu.async_copy(b_hbm.at[idxd], brows, sem)
            ca.wait()
            cb.wait()

            def group_body(g, inner):
                dd16 = ddb[pl.ds(g * 16, 16)]
                for lane in range(16):
                    e = g * 16 + lane
                    ddv = lax.broadcast(dd16[lane], (16,))
                    for j in range(H // 16):
                        sl = pl.ds(16 * j, 16)
                        trows[e, sl] = arows[e, sl] + brows[e, sl] + ddv * wvs[j]
                return inner

            lax.fori_loop(0, SC_EBLK // 16, group_body, 0)
            pltpu.sync_copy(trows, t_hbm.at[pl.ds(base, SC_EBLK)])
            return carry

        lax.fori_loop(0, chunks, chunk_body, 0)

    return k


@jax.jit
def _sc_gather_t(a, b, src, dst, dd, wd):
    return _make_sc_gather(src.shape[0])(a, b, src, dst, dd, wd)


def _round_bf16(v):
    u = jax.lax.bitcast_convert_type(v, jnp.uint32)
    r = (u + jnp.uint32(0x7FFF) + ((u >> 16) & jnp.uint32(1))) & jnp.uint32(0xFFFF0000)
    return jax.lax.bitcast_convert_type(r, jnp.float32)


def kernel(h, pos, edge_index, emb, edge_w1, edge_b1, edge_w2, edge_b2,
           coord_w1, coord_b1, coord_w2, coord_b2, node_w1, node_b1,
           node_w2, node_b2):
    L = edge_w1.shape[0]
    src = edge_index[0]
    dst = edge_index[1]
    n = pos.shape[0]
    deg = jax.ops.segment_sum(jnp.ones((src.shape[0], 1), pos.dtype), src,
                              num_segments=n)
    deg = jnp.maximum(deg, 1.0)
    x = emb[h]
    pos0 = pos
    a, b = _ab_precompute(x, edge_w1[0][:H], edge_b1[0], edge_w1[0][H:2 * H])
    for l in range(L):
        wd = _round_bf16(edge_w1[l][2 * H])
        rel = pos[src] - pos[dst]
        d2 = jnp.sum(rel * rel, axis=-1, keepdims=True)
        t = _sc_gather_t(a, b, src, dst, _round_bf16(d2[:, 0]), wd)
        m, c = _edge_mlp(t, edge_w2[l], edge_b2[l], coord_w1[l], coord_b1[l],
                         coord_w2[l], coord_b2[l])
        pos = pos + jax.ops.segment_sum(rel * c, src, num_segments=n) / deg
        agg = jax.ops.segment_sum(m, src, num_segments=n)
        if l + 1 < L:
            u = _silu(jnp.concatenate([x, agg], axis=-1) @ node_w1[l] + node_b1[l])
            x = x + (u @ node_w2[l] + node_b2[l])
            a, b = _ab_precompute(x, edge_w1[l + 1][:H], edge_b1[l + 1],
                                  edge_w1[l + 1][H:2 * H])
    return pos - pos0
